# baseline (speedup 1.0000x reference)
import functools

import jax
import jax.numpy as jnp
from jax import lax
from jax.experimental import pallas as pl
from jax.experimental.pallas import tpu as pltpu
from jax.experimental.pallas import tpu_sc as plsc

N = 10000
E = 320000
D = 128
B = 64

NC = 2
NS = 16
NW = NC * NS
EPT = E // NW
CH = 80
NCHUNK = 128
IH = NCHUNK // 2
EPT_P = NCHUNK * CH
NA = N + NS
RPT = 624
RPT_LAST = N - 15 * RPT


def _seg_sum_sc(out_nd, src3, dst3):
    mesh = plsc.VectorSubcoreMesh(core_axis_name="c", subcore_axis_name="s")

    @functools.partial(
        pl.kernel,
        out_type=[
            jax.ShapeDtypeStruct((N, D), jnp.float32),
            jax.ShapeDtypeStruct((N, D), jnp.float32),
        ],
        mesh=mesh,
        scratch_types=[
            pltpu.VMEM((IH, CH), jnp.int32),
            pltpu.VMEM((IH, CH), jnp.int32),
            pltpu.VMEM((CH, D), jnp.float32),
            pltpu.VMEM((CH, D), jnp.float32),
            pltpu.VMEM_SHARED((NA, D), jnp.float32),
            pltpu.SemaphoreType.DMA,
            pltpu.SemaphoreType.DMA,
        ],
    )
    def k(out_hbm, src_hbm, dst_hbm, p0_hbm, p1_hbm,
          sidx, didx, rows, rows_b, acc, gsem_a, gsem_b):
        c = lax.axis_index("c")
        s = lax.axis_index("s")
        wid = c * NS + s

        zv = jnp.zeros((16,), jnp.float32)

        @pl.loop(0, CH)
        def _(i):
            @pl.loop(0, D, step=16)
            def _(j):
                rows[i, pl.ds(j, 16)] = zv

        zbase = pl.multiple_of(s * RPT, 8)

        @pl.loop(0, RPT // CH)
        def _(kk):
            pltpu.sync_copy(rows, acc.at[pl.ds(zbase + kk * CH, CH)])

        @pl.when(s < NS - 1)
        def _():
            pltpu.sync_copy(rows.at[pl.ds(0, RPT - (RPT // CH) * CH)],
                            acc.at[pl.ds(zbase + (RPT // CH) * CH,
                                         RPT - (RPT // CH) * CH)])

        @pl.when(s == NS - 1)
        def _():
            pltpu.sync_copy(rows,
                            acc.at[pl.ds(15 * RPT + (RPT // CH) * CH, CH)])

        plsc.subcore_barrier()

        for half in range(2):
            pltpu.sync_copy(src_hbm.at[wid].at[pl.ds(half * IH, IH)], sidx)
            pltpu.sync_copy(dst_hbm.at[wid].at[pl.ds(half * IH, IH)], didx)

            pltpu.async_copy(out_hbm.at[sidx.at[0]], rows, gsem_a)
            pltpu.async_copy(out_hbm.at[sidx.at[1]], rows_b, gsem_b)

            @pl.loop(0, IH - 2, step=2)
            def _(t):
                pltpu.make_async_copy(out_hbm.at[sidx.at[t]], rows,
                                      gsem_a).wait()
                pltpu.sync_copy(rows, acc.at[didx.at[t]], add=True)
                pltpu.async_copy(out_hbm.at[sidx.at[t + 2]], rows, gsem_a)
                pltpu.make_async_copy(out_hbm.at[sidx.at[t + 1]], rows_b,
                                      gsem_b).wait()
                pltpu.sync_copy(rows_b, acc.at[didx.at[t + 1]], add=True)
                pltpu.async_copy(out_hbm.at[sidx.at[t + 3]], rows_b, gsem_b)

            pltpu.make_async_copy(out_hbm.at[sidx.at[IH - 2]], rows,
                                  gsem_a).wait()
            pltpu.sync_copy(rows, acc.at[didx.at[IH - 2]], add=True)
            pltpu.make_async_copy(out_hbm.at[sidx.at[IH - 1]], rows_b,
                                  gsem_b).wait()
            pltpu.sync_copy(rows_b, acc.at[didx.at[IH - 1]], add=True)

        plsc.subcore_barrier()

        wbase = pl.multiple_of(s * RPT, 8)

        @pl.when((c == 0) & (s < NS - 1))
        def _():
            pltpu.sync_copy(acc.at[pl.ds(wbase, RPT)],
                            p0_hbm.at[pl.ds(wbase, RPT)])

        @pl.when((c == 0) & (s == NS - 1))
        def _():
            pltpu.sync_copy(acc.at[pl.ds(15 * RPT, RPT_LAST)],
                            p0_hbm.at[pl.ds(15 * RPT, RPT_LAST)])

        @pl.when((c == 1) & (s < NS - 1))
        def _():
            pltpu.sync_copy(acc.at[pl.ds(wbase, RPT)],
                            p1_hbm.at[pl.ds(wbase, RPT)])

        @pl.when((c == 1) & (s == NS - 1))
        def _():
            pltpu.sync_copy(acc.at[pl.ds(15 * RPT, RPT_LAST)],
                            p1_hbm.at[pl.ds(15 * RPT, RPT_LAST)])

    return k(out_nd, src3, dst3)


_RB = 1000


def _lin0_tc(x, w0t, b0r):
    def body(x_ref, w_ref, b_ref, o_ref):
        o_ref[...] = jnp.maximum(
            jnp.dot(x_ref[...], w_ref[...], preferred_element_type=jnp.float32)
            + b_ref[...], 0.0)

    return pl.pallas_call(
        body,
        grid=(N // _RB,),
        in_specs=[
            pl.BlockSpec((_RB, D), lambda i: (i, 0)),
            pl.BlockSpec((D, D), lambda i: (0, 0)),
            pl.BlockSpec((1, D), lambda i: (0, 0)),
        ],
        out_specs=pl.BlockSpec((_RB, D), lambda i: (i, 0)),
        out_shape=jax.ShapeDtypeStruct((N, D), jnp.float32),
        compiler_params=pltpu.CompilerParams(
            dimension_semantics=("parallel",)),
    )(x, w0t, b0r)


def _gin_layer_tc(out, p0, p1, w1t, c1r, w2t, c2r, wiht, bihr, whht, bhhr):
    def body(o_ref, p0_ref, p1_ref, w1, c1_, w2, c2_, wih, bih_, whh, bhh_,
             newh_ref):
        hh = o_ref[...]
        z = hh + p0_ref[...] + p1_ref[...]
        t = jnp.maximum(
            jnp.dot(z, w1[...], preferred_element_type=jnp.float32) + c1_[...],
            0.0)
        m = jnp.maximum(
            jnp.dot(t, w2[...], preferred_element_type=jnp.float32) + c2_[...],
            0.0)
        gi = jnp.dot(m, wih[...], preferred_element_type=jnp.float32) + bih_[...]
        gh = jnp.dot(hh, whh[...], preferred_element_type=jnp.float32) + bhh_[...]
        r = jax.nn.sigmoid(gi[:, :D] + gh[:, :D])
        zz = jax.nn.sigmoid(gi[:, D:2 * D] + gh[:, D:2 * D])
        n = jnp.tanh(gi[:, 2 * D:] + r * gh[:, 2 * D:])
        newh_ref[...] = (1.0 - zz) * n + zz * hh

    full = lambda shape: pl.BlockSpec(shape, lambda i: (0, 0))
    return pl.pallas_call(
        body,
        grid=(N // _RB,),
        in_specs=[
            pl.BlockSpec((_RB, D), lambda i: (i, 0)),
            pl.BlockSpec((_RB, D), lambda i: (i, 0)),
            pl.BlockSpec((_RB, D), lambda i: (i, 0)),
            full((D, D)), full((1, D)),
            full((D, D)), full((1, D)),
            full((D, 3 * D)), full((1, 3 * D)),
            full((D, 3 * D)), full((1, 3 * D)),
        ],
        out_specs=pl.BlockSpec((_RB, D), lambda i: (i, 0)),
        out_shape=jax.ShapeDtypeStruct((N, D), jnp.float32),
        compiler_params=pltpu.CompilerParams(
            dimension_semantics=("parallel",)),
    )(out, p0, p1, w1t, c1r, w2t, c2r, wiht, bihr, whht, bhhr)


def _set2set_tc(out, batch_row, wiht, whht, bihr, bhhr):
    def body(o_ref, b_ref, wih, whh, bih, bhh, q_ref):
        ot = o_ref[...]
        br = b_ref[...]
        ids = lax.broadcasted_iota(jnp.int32, (B, N), 0)
        maskb = ids == br
        qh = jnp.zeros((B, D), jnp.float32)
        qc = jnp.zeros((B, D), jnp.float32)
        q_star = jnp.zeros((B, 2 * D), jnp.float32)
        for _ in range(3):
            gates = (
                jnp.dot(q_star, wih[...], preferred_element_type=jnp.float32)
                + bih[...]
                + jnp.dot(qh, whh[...], preferred_element_type=jnp.float32)
                + bhh[...])
            ii = jax.nn.sigmoid(gates[:, :D])
            ff = jax.nn.sigmoid(gates[:, D:2 * D])
            gg = jnp.tanh(gates[:, 2 * D:3 * D])
            oo = jax.nn.sigmoid(gates[:, 3 * D:])
            qc = ff * qc + ii * gg
            qh = oo * jnp.tanh(qc)
            eb = lax.dot_general(qh, ot, (((1,), (1,)), ((), ())),
                                 preferred_element_type=jnp.float32)
            emax = jnp.max(jnp.where(maskb, eb, -1e30), axis=1, keepdims=True)
            emax = jnp.where(emax > -1e29, emax, 0.0)
            ex = jnp.exp(jnp.where(maskb, eb - emax, -1e30))
            denom = jnp.sum(ex, axis=1, keepdims=True)
            a = ex / (denom + 1e-16)
            r = jnp.dot(a, ot, preferred_element_type=jnp.float32)
            q_star = jnp.concatenate([qh, r], axis=1)
        q_ref[...] = q_star

    return pl.pallas_call(
        body,
        out_shape=jax.ShapeDtypeStruct((B, 2 * D), jnp.float32),
    )(out, batch_row, wiht, whht, bihr, bhhr)


def _pad_edges(idx_row, junk_per_tile):
    per_tile = idx_row.reshape(NW, EPT)
    if EPT_P == EPT:
        return per_tile.reshape(NW, NCHUNK, CH)
    if junk_per_tile:
        fill = N + (jnp.arange(NW, dtype=jnp.int32) % NS)
        pad = jnp.broadcast_to(fill[:, None], (NW, EPT_P - EPT))
    else:
        pad = jnp.zeros((NW, EPT_P - EPT), dtype=jnp.int32)
    return jnp.concatenate([per_tile, pad], axis=1).reshape(NW, NCHUNK, CH)


def kernel(x, edge_index, batch, W0, b0, gru_Wih, gru_Whh, gru_bih, gru_bhh,
           W1, c1, W2, c2, ls_Wih, ls_Whh, ls_bih, ls_bhh):
    src3 = _pad_edges(edge_index[0], False)
    dst3 = _pad_edges(edge_index[1], True)
    batch_row = batch.reshape(1, N)

    out = _lin0_tc(x, W0.T, b0.reshape(1, D))
    for _ in range(3):
        p0, p1 = _seg_sum_sc(out, src3, dst3)
        out = _gin_layer_tc(out, p0, p1,
                            W1.T, c1.reshape(1, D), W2.T, c2.reshape(1, D),
                            gru_Wih.T, gru_bih.reshape(1, 3 * D),
                            gru_Whh.T, gru_bhh.reshape(1, 3 * D))
    q_star = _set2set_tc(out, batch_row,
                         ls_Wih.T, ls_Whh.T,
                         ls_bih.reshape(1, 4 * D), ls_bhh.reshape(1, 4 * D))
    return (q_star, out)


# speedup vs baseline: 1.8451x; 1.8451x over previous
import functools

import jax
import jax.numpy as jnp
from jax import lax
from jax.experimental import pallas as pl
from jax.experimental.pallas import tpu as pltpu
from jax.experimental.pallas import tpu_sc as plsc

N = 10000
E = 320000
D = 128
B = 64

NC = 2
NS = 16
NW = NC * NS
EPT = E // NW
CH = 80
NCHUNK = EPT // CH
EPT_P = NCHUNK * CH
NA = N + NS
RPT = 624
RPT_LAST = N - 15 * RPT


def _seg_sum_sc(out_nd, src3, dst3):
    mesh = plsc.VectorSubcoreMesh(core_axis_name="c", subcore_axis_name="s")

    @functools.partial(
        pl.kernel,
        out_type=[
            jax.ShapeDtypeStruct((N, D), jnp.float32),
            jax.ShapeDtypeStruct((N, D), jnp.float32),
        ],
        mesh=mesh,
        scratch_types=[
            pltpu.VMEM((NCHUNK, CH), jnp.int32),
            pltpu.VMEM((NCHUNK, CH), jnp.int32),
            pltpu.VMEM((CH, D), jnp.float32),
            pltpu.VMEM_SHARED((NA, D), jnp.float32),
        ],
    )
    def k(out_hbm, src_hbm, dst_hbm, p0_hbm, p1_hbm, sidx, didx, rows, acc):
        c = lax.axis_index("c")
        s = lax.axis_index("s")
        wid = c * NS + s

        zv = jnp.zeros((16,), jnp.float32)

        @pl.loop(0, CH)
        def _(i):
            @pl.loop(0, D, step=16)
            def _(j):
                rows[i, pl.ds(j, 16)] = zv

        zbase = pl.multiple_of(s * RPT, 8)

        @pl.loop(0, RPT // CH)
        def _(kk):
            pltpu.sync_copy(rows, acc.at[pl.ds(zbase + kk * CH, CH)])

        @pl.when(s < NS - 1)
        def _():
            pltpu.sync_copy(rows.at[pl.ds(0, RPT - (RPT // CH) * CH)],
                            acc.at[pl.ds(zbase + (RPT // CH) * CH,
                                         RPT - (RPT // CH) * CH)])

        @pl.when(s == NS - 1)
        def _():
            pltpu.sync_copy(rows,
                            acc.at[pl.ds(15 * RPT + (RPT // CH) * CH, CH)])

        plsc.subcore_barrier()

        pltpu.sync_copy(src_hbm.at[wid], sidx)
        pltpu.sync_copy(dst_hbm.at[wid], didx)

        @pl.loop(0, NCHUNK)
        def _(j):
            pltpu.sync_copy(out_hbm.at[sidx.at[j]], rows)
            pltpu.sync_copy(rows, acc.at[didx.at[j]], add=True)

        plsc.subcore_barrier()

        wbase = pl.multiple_of(s * RPT, 8)

        @pl.when((c == 0) & (s < NS - 1))
        def _():
            pltpu.sync_copy(acc.at[pl.ds(wbase, RPT)],
                            p0_hbm.at[pl.ds(wbase, RPT)])

        @pl.when((c == 0) & (s == NS - 1))
        def _():
            pltpu.sync_copy(acc.at[pl.ds(15 * RPT, RPT_LAST)],
                            p0_hbm.at[pl.ds(15 * RPT, RPT_LAST)])

        @pl.when((c == 1) & (s < NS - 1))
        def _():
            pltpu.sync_copy(acc.at[pl.ds(wbase, RPT)],
                            p1_hbm.at[pl.ds(wbase, RPT)])

        @pl.when((c == 1) & (s == NS - 1))
        def _():
            pltpu.sync_copy(acc.at[pl.ds(15 * RPT, RPT_LAST)],
                            p1_hbm.at[pl.ds(15 * RPT, RPT_LAST)])

    return k(out_nd, src3, dst3)


_RB = 1000


def _lin0_tc(x, w0t, b0r):
    def body(x_ref, w_ref, b_ref, o_ref):
        o_ref[...] = jnp.maximum(
            jnp.dot(x_ref[...], w_ref[...], preferred_element_type=jnp.float32)
            + b_ref[...], 0.0)

    return pl.pallas_call(
        body,
        grid=(N // _RB,),
        in_specs=[
            pl.BlockSpec((_RB, D), lambda i: (i, 0)),
            pl.BlockSpec((D, D), lambda i: (0, 0)),
            pl.BlockSpec((1, D), lambda i: (0, 0)),
        ],
        out_specs=pl.BlockSpec((_RB, D), lambda i: (i, 0)),
        out_shape=jax.ShapeDtypeStruct((N, D), jnp.float32),
        compiler_params=pltpu.CompilerParams(
            dimension_semantics=("parallel",)),
    )(x, w0t, b0r)


def _gin_layer_tc(out, p0, p1, w1t, c1r, w2t, c2r, wiht, bihr, whht, bhhr):
    def body(o_ref, p0_ref, p1_ref, w1, c1_, w2, c2_, wih, bih_, whh, bhh_,
             newh_ref):
        hh = o_ref[...]
        z = hh + p0_ref[...] + p1_ref[...]
        t = jnp.maximum(
            jnp.dot(z, w1[...], preferred_element_type=jnp.float32) + c1_[...],
            0.0)
        m = jnp.maximum(
            jnp.dot(t, w2[...], preferred_element_type=jnp.float32) + c2_[...],
            0.0)
        gi = jnp.dot(m, wih[...], preferred_element_type=jnp.float32) + bih_[...]
        gh = jnp.dot(hh, whh[...], preferred_element_type=jnp.float32) + bhh_[...]
        r = jax.nn.sigmoid(gi[:, :D] + gh[:, :D])
        zz = jax.nn.sigmoid(gi[:, D:2 * D] + gh[:, D:2 * D])
        n = jnp.tanh(gi[:, 2 * D:] + r * gh[:, 2 * D:])
        newh_ref[...] = (1.0 - zz) * n + zz * hh

    full = lambda shape: pl.BlockSpec(shape, lambda i: (0, 0))
    return pl.pallas_call(
        body,
        grid=(N // _RB,),
        in_specs=[
            pl.BlockSpec((_RB, D), lambda i: (i, 0)),
            pl.BlockSpec((_RB, D), lambda i: (i, 0)),
            pl.BlockSpec((_RB, D), lambda i: (i, 0)),
            full((D, D)), full((1, D)),
            full((D, D)), full((1, D)),
            full((D, 3 * D)), full((1, 3 * D)),
            full((D, 3 * D)), full((1, 3 * D)),
        ],
        out_specs=pl.BlockSpec((_RB, D), lambda i: (i, 0)),
        out_shape=jax.ShapeDtypeStruct((N, D), jnp.float32),
        compiler_params=pltpu.CompilerParams(
            dimension_semantics=("parallel",)),
    )(out, p0, p1, w1t, c1r, w2t, c2r, wiht, bihr, whht, bhhr)


def _set2set_tc(out, batch_row, wiht, whht, bihr, bhhr):
    def body(o_ref, b_ref, wih, whh, bih, bhh, q_ref):
        ot = o_ref[...]
        br = b_ref[...]
        ids = lax.broadcasted_iota(jnp.int32, (B, N), 0)
        maskb = ids == br
        qh = jnp.zeros((B, D), jnp.float32)
        qc = jnp.zeros((B, D), jnp.float32)
        q_star = jnp.zeros((B, 2 * D), jnp.float32)
        for _ in range(3):
            gates = (
                jnp.dot(q_star, wih[...], preferred_element_type=jnp.float32)
                + bih[...]
                + jnp.dot(qh, whh[...], preferred_element_type=jnp.float32)
                + bhh[...])
            ii = jax.nn.sigmoid(gates[:, :D])
            ff = jax.nn.sigmoid(gates[:, D:2 * D])
            gg = jnp.tanh(gates[:, 2 * D:3 * D])
            oo = jax.nn.sigmoid(gates[:, 3 * D:])
            qc = ff * qc + ii * gg
            qh = oo * jnp.tanh(qc)
            eb = lax.dot_general(qh, ot, (((1,), (1,)), ((), ())),
                                 preferred_element_type=jnp.float32)
            emax = jnp.max(jnp.where(maskb, eb, -1e30), axis=1, keepdims=True)
            emax = jnp.where(emax > -1e29, emax, 0.0)
            ex = jnp.exp(jnp.where(maskb, eb - emax, -1e30))
            denom = jnp.sum(ex, axis=1, keepdims=True)
            a = ex / (denom + 1e-16)
            r = jnp.dot(a, ot, preferred_element_type=jnp.float32)
            q_star = jnp.concatenate([qh, r], axis=1)
        q_ref[...] = q_star

    return pl.pallas_call(
        body,
        out_shape=jax.ShapeDtypeStruct((B, 2 * D), jnp.float32),
    )(out, batch_row, wiht, whht, bihr, bhhr)


def _pad_edges(idx_row, junk_per_tile):
    per_tile = idx_row.reshape(NW, EPT)
    if EPT_P == EPT:
        return per_tile.reshape(NW, NCHUNK, CH)
    if junk_per_tile:
        fill = N + (jnp.arange(NW, dtype=jnp.int32) % NS)
        pad = jnp.broadcast_to(fill[:, None], (NW, EPT_P - EPT))
    else:
        pad = jnp.zeros((NW, EPT_P - EPT), dtype=jnp.int32)
    return jnp.concatenate([per_tile, pad], axis=1).reshape(NW, NCHUNK, CH)


def kernel(x, edge_index, batch, W0, b0, gru_Wih, gru_Whh, gru_bih, gru_bhh,
           W1, c1, W2, c2, ls_Wih, ls_Whh, ls_bih, ls_bhh):
    src3 = _pad_edges(edge_index[0], False)
    dst3 = _pad_edges(edge_index[1], True)
    batch_row = batch.reshape(1, N)

    out = _lin0_tc(x, W0.T, b0.reshape(1, D))
    for _ in range(3):
        p0, p1 = _seg_sum_sc(out, src3, dst3)
        out = _gin_layer_tc(out, p0, p1,
                            W1.T, c1.reshape(1, D), W2.T, c2.reshape(1, D),
                            gru_Wih.T, gru_bih.reshape(1, 3 * D),
                            gru_Whh.T, gru_bhh.reshape(1, 3 * D))
    q_star = _set2set_tc(out, batch_row,
                         ls_Wih.T, ls_Whh.T,
                         ls_bih.reshape(1, 4 * D), ls_bhh.reshape(1, 4 * D))
    return (q_star, out)


# speedup vs baseline: 1.8457x; 1.0004x over previous
import functools

import jax
import jax.numpy as jnp
from jax import lax
from jax.experimental import pallas as pl
from jax.experimental.pallas import tpu as pltpu
from jax.experimental.pallas import tpu_sc as plsc

N = 10000
E = 320000
D = 128
B = 64

NC = 2
NS = 16
NW = NC * NS
EPT = E // NW
CH = 80
NCHUNK = EPT // CH
RPT = 624
RPT_LAST = N - 15 * RPT


def _seg_sum_sc(out_nd, src3, dst3):
    mesh = plsc.VectorSubcoreMesh(core_axis_name="c", subcore_axis_name="s")

    @functools.partial(
        pl.kernel,
        out_type=[
            jax.ShapeDtypeStruct((N, D), jnp.float32),
            jax.ShapeDtypeStruct((N, D), jnp.float32),
        ],
        mesh=mesh,
        scratch_types=[
            pltpu.VMEM((NCHUNK, CH), jnp.int32),
            pltpu.VMEM((NCHUNK, CH), jnp.int32),
            pltpu.VMEM((CH, D), jnp.float32),
            pltpu.VMEM_SHARED((N, D), jnp.float32),
        ],
    )
    def k(out_hbm, src_hbm, dst_hbm, p0_hbm, p1_hbm, sidx, didx, rows, acc):
        c = lax.axis_index("c")
        s = lax.axis_index("s")
        wid = c * NS + s

        zv = jnp.zeros((16,), jnp.float32)

        @pl.loop(0, CH)
        def _(i):
            @pl.loop(0, D, step=16)
            def _(j):
                rows[i, pl.ds(j, 16)] = zv

        zbase = pl.multiple_of(s * RPT, 8)

        @pl.loop(0, RPT // CH)
        def _(kk):
            pltpu.sync_copy(rows, acc.at[pl.ds(zbase + kk * CH, CH)])

        @pl.when(s < NS - 1)
        def _():
            pltpu.sync_copy(rows.at[pl.ds(0, RPT - (RPT // CH) * CH)],
                            acc.at[pl.ds(zbase + (RPT // CH) * CH,
                                         RPT - (RPT // CH) * CH)])

        @pl.when(s == NS - 1)
        def _():
            pltpu.sync_copy(rows,
                            acc.at[pl.ds(15 * RPT + (RPT // CH) * CH, CH)])

        plsc.subcore_barrier()

        pltpu.sync_copy(src_hbm.at[wid], sidx)
        pltpu.sync_copy(dst_hbm.at[wid], didx)

        @pl.loop(0, NCHUNK)
        def _(j):
            pltpu.sync_copy(out_hbm.at[sidx.at[j]], rows)
            pltpu.sync_copy(rows, acc.at[didx.at[j]], add=True)

        plsc.subcore_barrier()

        wbase = pl.multiple_of(s * RPT, 8)

        @pl.when((c == 0) & (s < NS - 1))
        def _():
            pltpu.sync_copy(acc.at[pl.ds(wbase, RPT)],
                            p0_hbm.at[pl.ds(wbase, RPT)])

        @pl.when((c == 0) & (s == NS - 1))
        def _():
            pltpu.sync_copy(acc.at[pl.ds(15 * RPT, RPT_LAST)],
                            p0_hbm.at[pl.ds(15 * RPT, RPT_LAST)])

        @pl.when((c == 1) & (s < NS - 1))
        def _():
            pltpu.sync_copy(acc.at[pl.ds(wbase, RPT)],
                            p1_hbm.at[pl.ds(wbase, RPT)])

        @pl.when((c == 1) & (s == NS - 1))
        def _():
            pltpu.sync_copy(acc.at[pl.ds(15 * RPT, RPT_LAST)],
                            p1_hbm.at[pl.ds(15 * RPT, RPT_LAST)])

    return k(out_nd, src3, dst3)


_RB = 1000


def _lin0_tc(x, w0t, b0r):
    def body(x_ref, w_ref, b_ref, o_ref):
        o_ref[...] = jnp.maximum(
            jnp.dot(x_ref[...], w_ref[...], preferred_element_type=jnp.float32)
            + b_ref[...], 0.0)

    return pl.pallas_call(
        body,
        grid=(N // _RB,),
        in_specs=[
            pl.BlockSpec((_RB, D), lambda i: (i, 0)),
            pl.BlockSpec((D, D), lambda i: (0, 0)),
            pl.BlockSpec((1, D), lambda i: (0, 0)),
        ],
        out_specs=pl.BlockSpec((_RB, D), lambda i: (i, 0)),
        out_shape=jax.ShapeDtypeStruct((N, D), jnp.float32),
        compiler_params=pltpu.CompilerParams(
            dimension_semantics=("parallel",)),
    )(x, w0t, b0r)


def _gin_layer_tc(out, p0, p1, w1t, c1r, w2t, c2r, wiht, bihr, whht, bhhr):
    def body(o_ref, p0_ref, p1_ref, w1, c1_, w2, c2_, wih, bih_, whh, bhh_,
             newh_ref):
        hh = o_ref[...]
        z = hh + p0_ref[...] + p1_ref[...]
        t = jnp.maximum(
            jnp.dot(z, w1[...], preferred_element_type=jnp.float32) + c1_[...],
            0.0)
        m = jnp.maximum(
            jnp.dot(t, w2[...], preferred_element_type=jnp.float32) + c2_[...],
            0.0)
        gi = jnp.dot(m, wih[...], preferred_element_type=jnp.float32) + bih_[...]
        gh = jnp.dot(hh, whh[...], preferred_element_type=jnp.float32) + bhh_[...]
        r = jax.nn.sigmoid(gi[:, :D] + gh[:, :D])
        zz = jax.nn.sigmoid(gi[:, D:2 * D] + gh[:, D:2 * D])
        n = jnp.tanh(gi[:, 2 * D:] + r * gh[:, 2 * D:])
        newh_ref[...] = (1.0 - zz) * n + zz * hh

    full = lambda shape: pl.BlockSpec(shape, lambda i: (0, 0))
    return pl.pallas_call(
        body,
        grid=(N // _RB,),
        in_specs=[
            pl.BlockSpec((_RB, D), lambda i: (i, 0)),
            pl.BlockSpec((_RB, D), lambda i: (i, 0)),
            pl.BlockSpec((_RB, D), lambda i: (i, 0)),
            full((D, D)), full((1, D)),
            full((D, D)), full((1, D)),
            full((D, 3 * D)), full((1, 3 * D)),
            full((D, 3 * D)), full((1, 3 * D)),
        ],
        out_specs=pl.BlockSpec((_RB, D), lambda i: (i, 0)),
        out_shape=jax.ShapeDtypeStruct((N, D), jnp.float32),
        compiler_params=pltpu.CompilerParams(
            dimension_semantics=("parallel",)),
    )(out, p0, p1, w1t, c1r, w2t, c2r, wiht, bihr, whht, bhhr)


def _set2set_tc(out, batch_row, wiht, whht, bihr, bhhr):
    def body(o_ref, b_ref, wih, whh, bih, bhh, q_ref):
        ot = o_ref[...]
        br = b_ref[...]
        ids = lax.broadcasted_iota(jnp.int32, (B, N), 0)
        maskb = ids == br
        qh = jnp.zeros((B, D), jnp.float32)
        qc = jnp.zeros((B, D), jnp.float32)
        q_star = jnp.zeros((B, 2 * D), jnp.float32)
        for _ in range(3):
            gates = (
                jnp.dot(q_star, wih[...], preferred_element_type=jnp.float32)
                + bih[...]
                + jnp.dot(qh, whh[...], preferred_element_type=jnp.float32)
                + bhh[...])
            ii = jax.nn.sigmoid(gates[:, :D])
            ff = jax.nn.sigmoid(gates[:, D:2 * D])
            gg = jnp.tanh(gates[:, 2 * D:3 * D])
            oo = jax.nn.sigmoid(gates[:, 3 * D:])
            qc = ff * qc + ii * gg
            qh = oo * jnp.tanh(qc)
            eb = lax.dot_general(qh, ot, (((1,), (1,)), ((), ())),
                                 preferred_element_type=jnp.float32)
            emax = jnp.max(jnp.where(maskb, eb, -1e30), axis=1, keepdims=True)
            emax = jnp.where(emax > -1e29, emax, 0.0)
            ex = jnp.exp(jnp.where(maskb, eb - emax, -1e30))
            denom = jnp.sum(ex, axis=1, keepdims=True)
            a = ex / (denom + 1e-16)
            r = jnp.dot(a, ot, preferred_element_type=jnp.float32)
            q_star = jnp.concatenate([qh, r], axis=1)
        q_ref[...] = q_star

    return pl.pallas_call(
        body,
        out_shape=jax.ShapeDtypeStruct((B, 2 * D), jnp.float32),
    )(out, batch_row, wiht, whht, bihr, bhhr)


def kernel(x, edge_index, batch, W0, b0, gru_Wih, gru_Whh, gru_bih, gru_bhh,
           W1, c1, W2, c2, ls_Wih, ls_Whh, ls_bih, ls_bhh):
    src3 = edge_index[0].reshape(NW, NCHUNK, CH)
    dst3 = edge_index[1].reshape(NW, NCHUNK, CH)
    batch_row = batch.reshape(1, N)

    out = _lin0_tc(x, W0.T, b0.reshape(1, D))
    for _ in range(3):
        p0, p1 = _seg_sum_sc(out, src3, dst3)
        out = _gin_layer_tc(out, p0, p1,
                            W1.T, c1.reshape(1, D), W2.T, c2.reshape(1, D),
                            gru_Wih.T, gru_bih.reshape(1, 3 * D),
                            gru_Whh.T, gru_bhh.reshape(1, 3 * D))
    q_star = _set2set_tc(out, batch_row,
                         ls_Wih.T, ls_Whh.T,
                         ls_bih.reshape(1, 4 * D), ls_bhh.reshape(1, 4 * D))
    return (q_star, out)
